# baseline (speedup 1.0000x reference)
import functools

import jax
import jax.numpy as jnp
from jax import lax
from jax.experimental import pallas as pl
from jax.experimental.pallas import tpu as pltpu
from jax.experimental.pallas import tpu_sc as plsc

EMB = 128
B = 4096
L = 50

NC = 2
NS = 16
NW = NC * NS

ROWS = B * L
RPW = ROWS // NW
GATHER = 100
CHUNK = 2 * GATHER
NCH = RPW // CHUNK
NBUF = 4

_mesh = plsc.VectorSubcoreMesh(core_axis_name="c", subcore_axis_name="s")


@functools.partial(
    pl.kernel,
    mesh=_mesh,
    out_type=jax.ShapeDtypeStruct((ROWS, EMB), jnp.float32),
    scratch_types=[
        pltpu.VMEM((2 * NCH, GATHER), jnp.int32),
    ]
    + [pltpu.VMEM((CHUNK, EMB), jnp.float32) for _ in range(NBUF)]
    + [pltpu.SemaphoreType.DMA for _ in range(2 * NBUF)],
)
def _shifted_gather(idx_hbm, table_hbm, out_hbm, idx_v, *bufs_and_sems):
    bufs = bufs_and_sems[:NBUF]
    gsem = bufs_and_sems[NBUF : 2 * NBUF]
    osem = bufs_and_sems[2 * NBUF :]
    wid = lax.axis_index("s") * NC + lax.axis_index("c")
    pltpu.sync_copy(idx_hbm.at[wid], idx_v)
    zeros16 = jnp.zeros((16,), jnp.float32)

    def group(g, carry):
        for b in range(NBUF):
            j = g * NBUF + b
            base = wid * RPW + j * CHUNK

            @pl.when(g > 0)
            def _():
                pltpu.make_async_copy(
                    bufs[b], out_hbm.at[pl.ds(base, CHUNK)], osem[b]
                ).wait()

            pltpu.async_copy(
                table_hbm.at[idx_v.at[2 * j]], bufs[b].at[pl.ds(0, GATHER)], gsem[b]
            )
            pltpu.async_copy(
                table_hbm.at[idx_v.at[2 * j + 1]],
                bufs[b].at[pl.ds(GATHER, GATHER)],
                gsem[b],
            )

        for b in range(NBUF):
            j = g * NBUF + b
            base = wid * RPW + j * CHUNK
            pltpu.make_async_copy(
                table_hbm.at[idx_v.at[2 * j]], bufs[b].at[pl.ds(0, GATHER)], gsem[b]
            ).wait()
            pltpu.make_async_copy(
                table_hbm.at[idx_v.at[2 * j + 1]],
                bufs[b].at[pl.ds(GATHER, GATHER)],
                gsem[b],
            ).wait()
            for r in range(L - 1, CHUNK, L):
                for k in range(EMB // 16):
                    bufs[b][r, pl.ds(k * 16, 16)] = zeros16
            pltpu.async_copy(bufs[b], out_hbm.at[pl.ds(base, CHUNK)], osem[b])
        return carry

    lax.fori_loop(0, NCH // NBUF, group, 0)
    for b in range(NBUF):
        base = wid * RPW + (NCH - NBUF + b) * CHUNK
        pltpu.make_async_copy(
            bufs[b], out_hbm.at[pl.ds(base, CHUNK)], osem[b]
        ).wait()


def kernel(x, table):
    idx = jnp.concatenate(
        [x[:, 1:], jnp.zeros((B, 1), dtype=x.dtype)], axis=1
    ).astype(jnp.int32)
    idx = idx.reshape(NW, 2 * NCH, GATHER)
    out = _shifted_gather(idx, table)
    return out.reshape(B, L, EMB)


# speedup vs baseline: 1.8707x; 1.8707x over previous
import functools

import jax
import jax.numpy as jnp
from jax import lax
from jax.experimental import pallas as pl
from jax.experimental.pallas import tpu as pltpu
from jax.experimental.pallas import tpu_sc as plsc

EMB = 128
B = 4096
L = 50

NC = 2
NS = 16
NW = NC * NS

ROWS = B * L
RPW = ROWS // NW
BPW = B // NW
BPC = 4
CHUNK = BPC * L
NCH = RPW // CHUNK
NBUF = 4

_mesh = plsc.VectorSubcoreMesh(core_axis_name="c", subcore_axis_name="s")


@functools.partial(
    pl.kernel,
    mesh=_mesh,
    out_type=jax.ShapeDtypeStruct((ROWS, EMB), jnp.float32),
    scratch_types=[
        pltpu.VMEM((BPW, L), jnp.int32),
    ]
    + [pltpu.VMEM((CHUNK + 1, EMB), jnp.float32) for _ in range(NBUF)]
    + [pltpu.SemaphoreType.DMA for _ in range(2 * NBUF)],
)
def _shifted_gather(x_hbm, table_hbm, out_hbm, x_v, *bufs_and_sems):
    bufs = bufs_and_sems[:NBUF]
    gsem = bufs_and_sems[NBUF : 2 * NBUF]
    osem = bufs_and_sems[2 * NBUF :]
    wid = lax.axis_index("s") * NC + lax.axis_index("c")
    pltpu.sync_copy(x_hbm.at[pl.ds(wid * BPW, BPW)], x_v)
    zeros16 = jnp.zeros((16,), jnp.float32)

    def group(g, carry):
        for s in range(NBUF):
            j = g * NBUF + s
            base = wid * RPW + j * CHUNK

            @pl.when(g > 0)
            def _():
                pltpu.make_async_copy(
                    bufs[s].at[pl.ds(1, CHUNK)],
                    out_hbm.at[pl.ds(base, CHUNK)],
                    osem[s],
                ).wait()

            for k in range(BPC):
                pltpu.async_copy(
                    table_hbm.at[x_v.at[j * BPC + k]],
                    bufs[s].at[pl.ds(k * L, L)],
                    gsem[s],
                )

        for s in range(NBUF):
            j = g * NBUF + s
            base = wid * RPW + j * CHUNK
            for k in range(BPC):
                pltpu.make_async_copy(
                    table_hbm.at[x_v.at[j * BPC + k]],
                    bufs[s].at[pl.ds(k * L, L)],
                    gsem[s],
                ).wait()
            for r in range(L, CHUNK + 1, L):
                for k in range(EMB // 16):
                    bufs[s][r, pl.ds(k * 16, 16)] = zeros16
            pltpu.async_copy(
                bufs[s].at[pl.ds(1, CHUNK)], out_hbm.at[pl.ds(base, CHUNK)], osem[s]
            )
        return carry

    lax.fori_loop(0, NCH // NBUF, group, 0)
    for s in range(NBUF):
        base = wid * RPW + (NCH - NBUF + s) * CHUNK
        pltpu.make_async_copy(
            bufs[s].at[pl.ds(1, CHUNK)], out_hbm.at[pl.ds(base, CHUNK)], osem[s]
        ).wait()


def kernel(x, table):
    out = _shifted_gather(x.astype(jnp.int32), table)
    return out.reshape(B, L, EMB)


# speedup vs baseline: 3.3041x; 1.7662x over previous
import functools

import jax
import jax.numpy as jnp
from jax import lax
from jax.experimental import pallas as pl
from jax.experimental.pallas import tpu as pltpu
from jax.experimental.pallas import tpu_sc as plsc

EMB = 128
B = 4096
L = 50

NC = 2
NS = 16
NW = NC * NS

ROWS = B * L
RPW = ROWS // NW
BPW = B // NW
BPC = 4
CHUNK = BPC * L
NCH = RPW // CHUNK
NBUF = 4

_mesh = plsc.VectorSubcoreMesh(core_axis_name="c", subcore_axis_name="s")


@functools.partial(
    pl.kernel,
    mesh=_mesh,
    out_type=jax.ShapeDtypeStruct((B, L, EMB), jnp.float32),
    scratch_types=[
        pltpu.VMEM((BPW, L), jnp.int32),
    ]
    + [pltpu.VMEM((CHUNK + 1, EMB), jnp.float32) for _ in range(NBUF)]
    + [pltpu.SemaphoreType.DMA for _ in range(2 * NBUF)],
)
def _shifted_gather(x_hbm, table_hbm, out_hbm, x_v, *bufs_and_sems):
    bufs = bufs_and_sems[:NBUF]
    gsem = bufs_and_sems[NBUF : 2 * NBUF]
    osem = bufs_and_sems[2 * NBUF :]
    wid = lax.axis_index("s") * NC + lax.axis_index("c")
    pltpu.sync_copy(x_hbm.at[pl.ds(wid * BPW, BPW)], x_v)
    zeros16 = jnp.zeros((16,), jnp.float32)

    def group(g, carry):
        for s in range(NBUF):
            j = g * NBUF + s
            gb0 = wid * BPW + j * BPC

            @pl.when(g > 0)
            def _():
                for k in range(BPC):
                    pltpu.make_async_copy(
                        bufs[s].at[pl.ds(1 + k * L, L)], out_hbm.at[gb0 + k], osem[s]
                    ).wait()

            for k in range(BPC):
                pltpu.async_copy(
                    table_hbm.at[x_v.at[j * BPC + k]],
                    bufs[s].at[pl.ds(k * L, L)],
                    gsem[s],
                )

        for s in range(NBUF):
            j = g * NBUF + s
            gb0 = wid * BPW + j * BPC
            for k in range(BPC):
                pltpu.make_async_copy(
                    table_hbm.at[x_v.at[j * BPC + k]],
                    bufs[s].at[pl.ds(k * L, L)],
                    gsem[s],
                ).wait()
            for r in range(L, CHUNK + 1, L):
                for k in range(EMB // 16):
                    bufs[s][r, pl.ds(k * 16, 16)] = zeros16
            for k in range(BPC):
                pltpu.async_copy(
                    bufs[s].at[pl.ds(1 + k * L, L)], out_hbm.at[gb0 + k], osem[s]
                )
        return carry

    lax.fori_loop(0, NCH // NBUF, group, 0)
    for s in range(NBUF):
        gb0 = wid * BPW + (NCH - NBUF + s) * BPC
        for k in range(BPC):
            pltpu.make_async_copy(
                bufs[s].at[pl.ds(1 + k * L, L)], out_hbm.at[gb0 + k], osem[s]
            ).wait()


def kernel(x, table):
    return _shifted_gather(x.astype(jnp.int32), table)


# speedup vs baseline: 3.3278x; 1.0072x over previous
import functools

import jax
import jax.numpy as jnp
from jax import lax
from jax.experimental import pallas as pl
from jax.experimental.pallas import tpu as pltpu
from jax.experimental.pallas import tpu_sc as plsc

EMB = 128
B = 4096
L = 50

NC = 2
NS = 16
NW = NC * NS

ROWS = B * L
RPW = ROWS // NW
BPW = B // NW
BPC = 4
CHUNK = BPC * L
NCH = RPW // CHUNK
NBUF = 4

_mesh = plsc.VectorSubcoreMesh(core_axis_name="c", subcore_axis_name="s")


@functools.partial(
    pl.kernel,
    mesh=_mesh,
    out_type=jax.ShapeDtypeStruct((B, L, EMB), jnp.float32),
    scratch_types=[
        pltpu.VMEM((BPW, 128), jnp.int32),
    ]
    + [pltpu.VMEM((CHUNK + 1, EMB), jnp.float32) for _ in range(NBUF)]
    + [pltpu.SemaphoreType.DMA for _ in range(2 * NBUF)],
)
def _shifted_gather(x_hbm, table_hbm, out_hbm, x_v, *bufs_and_sems):
    bufs = bufs_and_sems[:NBUF]
    gsem = bufs_and_sems[NBUF : 2 * NBUF]
    osem = bufs_and_sems[2 * NBUF :]
    wid = lax.axis_index("s") * NC + lax.axis_index("c")
    pltpu.sync_copy(x_hbm.at[pl.ds(wid * BPW, BPW)], x_v)
    zeros16 = jnp.zeros((16,), jnp.float32)

    def group(g, carry):
        for s in range(NBUF):
            j = g * NBUF + s
            gb0 = wid * BPW + j * BPC

            @pl.when(g > 0)
            def _():
                for k in range(BPC):
                    pltpu.make_async_copy(
                        bufs[s].at[pl.ds(1 + k * L, L)], out_hbm.at[gb0 + k], osem[s]
                    ).wait()

            for k in range(BPC):
                pltpu.async_copy(
                    table_hbm.at[x_v.at[j * BPC + k, pl.ds(0, L)]],
                    bufs[s].at[pl.ds(k * L, L)],
                    gsem[s],
                )

        for s in range(NBUF):
            j = g * NBUF + s
            gb0 = wid * BPW + j * BPC
            for k in range(BPC):
                pltpu.make_async_copy(
                    table_hbm.at[x_v.at[j * BPC + k, pl.ds(0, L)]],
                    bufs[s].at[pl.ds(k * L, L)],
                    gsem[s],
                ).wait()
            for r in range(L, CHUNK + 1, L):
                for k in range(EMB // 16):
                    bufs[s][r, pl.ds(k * 16, 16)] = zeros16
            for k in range(BPC):
                pltpu.async_copy(
                    bufs[s].at[pl.ds(1 + k * L, L)], out_hbm.at[gb0 + k], osem[s]
                )
        return carry

    lax.fori_loop(0, NCH // NBUF, group, 0)
    for s in range(NBUF):
        gb0 = wid * BPW + (NCH - NBUF + s) * BPC
        for k in range(BPC):
            pltpu.make_async_copy(
                bufs[s].at[pl.ds(1 + k * L, L)], out_hbm.at[gb0 + k], osem[s]
            ).wait()


def kernel(x, table):
    xp = jnp.pad(x.astype(jnp.int32), ((0, 0), (0, 128 - L)))
    return _shifted_gather(xp, table)
